# baseline (speedup 1.0000x reference)
import functools

import jax
import jax.numpy as jnp
from jax import lax
from jax.experimental import pallas as pl
from jax.experimental.pallas import tpu as pltpu
from jax.experimental.pallas import tpu_sc as plsc

_B = 16384
_D = 64
_S = 2
_NC = 2
_NS = 16
_NW = _NC * _NS
_TOT = _S * _B
_RPW = _TOT // _NW
_CHUNK = 128
_NCH = _RPW // _CHUNK

def _sc_gather_body(table_hbm, idx_hbm, out_hbm, idx_v, rows_v, sem):
    wid = lax.axis_index("s") * _NC + lax.axis_index("c")
    pltpu.sync_copy(idx_hbm.at[wid], idx_v)
    copies = []
    for j in range(_NCH):
        copies.append(
            pltpu.async_copy(
                table_hbm.at[idx_v.at[j]],
                rows_v.at[pl.ds(j * _CHUNK, _CHUNK)],
                sem,
            )
        )
    for c in copies:
        c.wait()
    pltpu.sync_copy(rows_v, out_hbm.at[pl.ds(wid * _RPW, _RPW)])


@functools.cache
def _sc_gather():
    mesh = plsc.VectorSubcoreMesh(
        core_axis_name="c", subcore_axis_name="s", num_cores=_NC, num_subcores=_NS
    )
    return pl.kernel(
        _sc_gather_body,
        out_type=jax.ShapeDtypeStruct((_TOT, _D), jnp.float32),
        mesh=mesh,
        scratch_types=[
            pltpu.VMEM((_NCH, _CHUNK), jnp.int32),
            pltpu.VMEM((_RPW, _D), jnp.float32),
            pltpu.SemaphoreType.DMA,
        ],
        compiler_params=pltpu.CompilerParams(use_tc_tiling_on_sc=False),
    )


_BLK = 2048
_GRID = _B // _BLK


def _loss_body(center_ref, context_ref, neg_ref, out_ref):
    i = pl.program_id(0)

    @pl.when(i == 0)
    def _():
        out_ref[0, 0] = 0.0

    c = center_ref[...]
    pos = jnp.sum(c * context_ref[...], axis=1, keepdims=True)
    n0 = jnp.sum(c * neg_ref[0], axis=1, keepdims=True)
    n1 = jnp.sum(c * neg_ref[1], axis=1, keepdims=True)

    def bce(x, y):
        return jnp.maximum(x, 0.0) - x * y + jnp.log1p(jnp.exp(-jnp.abs(x)))

    chunk = jnp.sum(bce(pos, 1.0)) + jnp.sum(bce(n0, 0.0)) + jnp.sum(bce(n1, 0.0))
    tot = out_ref[0, 0] + chunk
    out_ref[0, 0] = jnp.where(i == _GRID - 1, tot * (1.0 / _B), tot)


def kernel(center, context, emb_table, neg_idx):
    idx = neg_idx.astype(jnp.int32).reshape(_NW, _NCH, _CHUNK)
    rows = _sc_gather()(emb_table, idx)
    neg = rows.reshape(_S, _B, _D)
    out = pl.pallas_call(
        _loss_body,
        grid=(_GRID,),
        in_specs=[
            pl.BlockSpec((_BLK, _D), lambda i: (i, 0)),
            pl.BlockSpec((_BLK, _D), lambda i: (i, 0)),
            pl.BlockSpec((_S, _BLK, _D), lambda i: (0, i, 0)),
        ],
        out_specs=pl.BlockSpec(memory_space=pltpu.SMEM),
        out_shape=jax.ShapeDtypeStruct((1, 1), jnp.float32),
    )(center, context, neg)
    return out[0, 0]
